# Initial kernel scaffold; baseline (speedup 1.0000x reference)
import jax
import jax.numpy as jnp
from jax.experimental import pallas as pl


def kernel(x, edge, a, I):
    raise NotImplementedError("write your pallas kernel here")


# speedup vs baseline: 13.3441x; 13.3441x over previous
import functools

import jax
import jax.numpy as jnp
from jax import lax
from jax.experimental import pallas as pl
from jax.experimental.pallas import tpu as pltpu
from jax.experimental.pallas import tpu_sc as plsc

ALPHA = 0.2
LANES = 16
NBLK = 2048


def _k0_body(w_ref, x_ref, alpha_ref, m_ref):
  i = pl.program_id(0)
  blk = lax.dot_general(
      w_ref[...], x_ref[...], (((1,), (1,)), ((), ())),
      preferred_element_type=jnp.float32)
  alpha_ref[...] = blk
  bm = jnp.broadcast_to(jnp.max(blk, axis=1, keepdims=True), m_ref.shape)

  @pl.when(i == 0)
  def _():
    m_ref[...] = bm

  @pl.when(i > 0)
  def _():
    m_ref[...] = jnp.maximum(m_ref[...], bm)


def _k2_body(part_ref, den_ref):
  den_ref[...] = jnp.sum(part_ref[...], axis=0)


def _make_k1(num_workers, epw, n_pad, heads):
  mesh = plsc.VectorSubcoreMesh(core_axis_name="c", subcore_axis_name="s", num_cores=2, num_subcores=16)

  @functools.partial(
      pl.kernel,
      mesh=mesh,
      compiler_params=pltpu.CompilerParams(needs_layout_passes=False),
      out_type=(
          jax.ShapeDtypeStruct((heads * num_workers * epw,), jnp.float32),
          jax.ShapeDtypeStruct((num_workers * heads * n_pad,), jnp.float32),
      ),
      scratch_types=[
          pltpu.VMEM((epw,), jnp.int32),
          pltpu.VMEM((epw,), jnp.int32),
          pltpu.VMEM((n_pad,), jnp.float32),
          pltpu.VMEM((n_pad,), jnp.float32),
          pltpu.VMEM((n_pad,), jnp.float32),
          pltpu.VMEM((epw,), jnp.float32),
          pltpu.VMEM((LANES,), jnp.float32),
      ],
  )
  def k1(edge_hbm, alpha_hbm, mb_hbm, eexp_hbm, part_hbm,
         src_v, dst_v, as_v, ad_v, den_v, p_v, m_v):
    e_total = num_workers * epw
    wid = lax.axis_index("s") * 2 + lax.axis_index("c")
    base = wid * epw
    pltpu.sync_copy(edge_hbm.at[pl.ds(base, epw)], src_v)
    pltpu.sync_copy(edge_hbm.at[pl.ds(e_total + base, epw)], dst_v)

    for h in range(heads):
      pltpu.sync_copy(alpha_hbm.at[pl.ds(h * n_pad, n_pad)], as_v)
      pltpu.sync_copy(alpha_hbm.at[pl.ds((heads + h) * n_pad, n_pad)], ad_v)
      pltpu.sync_copy(mb_hbm.at[pl.ds(h * LANES, LANES)], m_v)
      mvec = m_v[...]

      def zero_body(i, _):
        den_v[pl.ds(i * LANES, LANES)] = jnp.zeros((LANES,), jnp.float32)
        return 0

      lax.fori_loop(0, n_pad // LANES, zero_body, 0)

      def g_body(g, _):
        isrc = src_v[pl.ds(g * LANES, LANES)]
        idst = dst_v[pl.ds(g * LANES, LANES)]
        vs = plsc.load_gather(as_v, [isrc])
        vd = plsc.load_gather(ad_v, [idst])
        e = vs + vd
        e = jnp.where(e >= 0.0, e, ALPHA * e)
        p = jnp.exp(e - mvec)
        p_v[pl.ds(g * LANES, LANES)] = p
        plsc.addupdate_scatter(den_v, [idst], p)
        return 0

      lax.fori_loop(0, epw // LANES, g_body, 0)
      pltpu.sync_copy(p_v, eexp_hbm.at[pl.ds(h * e_total + base, epw)])
      pltpu.sync_copy(den_v, part_hbm.at[pl.ds((wid * heads + h) * n_pad, n_pad)])

  return k1


def _make_k3(num_workers, epw, n_pad, heads):
  mesh = plsc.VectorSubcoreMesh(core_axis_name="c", subcore_axis_name="s", num_cores=2, num_subcores=16)

  @functools.partial(
      pl.kernel,
      mesh=mesh,
      compiler_params=pltpu.CompilerParams(needs_layout_passes=False),
      out_type=jax.ShapeDtypeStruct((num_workers * epw * heads,), jnp.float32),
      scratch_types=[
          pltpu.VMEM((epw,), jnp.int32),
          pltpu.VMEM((n_pad,), jnp.float32),
          pltpu.VMEM((epw,), jnp.float32),
          pltpu.VMEM((epw * heads,), jnp.float32),
      ],
  )
  def k3(edge_hbm, eexp_hbm, den_hbm, att_hbm, dst_v, den_v, p_v, out_v):
    e_total = num_workers * epw
    wid = lax.axis_index("s") * 2 + lax.axis_index("c")
    base = wid * epw
    pltpu.sync_copy(edge_hbm.at[pl.ds(e_total + base, epw)], dst_v)
    lanes = lax.iota(jnp.int32, LANES)

    for h in range(heads):
      pltpu.sync_copy(den_hbm.at[pl.ds(h * n_pad, n_pad)], den_v)
      pltpu.sync_copy(eexp_hbm.at[pl.ds(h * e_total + base, epw)], p_v)
      hvec = jnp.full((LANES,), h, jnp.int32)

      def g_body(g, _):
        idst = dst_v[pl.ds(g * LANES, LANES)]
        d = plsc.load_gather(den_v, [idst])
        p = p_v[pl.ds(g * LANES, LANES)]
        o = p / (d + 1e-16)
        plsc.store_scatter(out_v, [(g * LANES + lanes) * heads + hvec], o)
        return 0

      lax.fori_loop(0, epw // LANES, g_body, 0)
    pltpu.sync_copy(out_v, att_hbm.at[pl.ds(base * heads, epw * heads)])

  return k3


def kernel(x, edge, a, I):
  n, f = x.shape
  e_total = edge.shape[1]
  dk = a.shape[0] // 2
  heads = f // dk
  num_workers = 32
  epw = e_total // num_workers
  n_pad = ((n + NBLK - 1) // NBLK) * NBLK

  eye_h = jnp.eye(heads, dtype=x.dtype)
  w_src = jnp.kron(eye_h, a[:dk, 0, 0][None, :])
  w_dst = jnp.kron(eye_h, a[dk:, 0, 0][None, :])
  w = jnp.concatenate([w_src, w_dst], axis=0)
  w_eff = w @ I.T
  xp = jnp.pad(x, ((0, n_pad - n), (0, 0)))

  grid = n_pad // NBLK
  alpha, mout = pl.pallas_call(
      _k0_body,
      grid=(grid,),
      in_specs=[
          pl.BlockSpec((2 * heads, f), lambda i: (0, 0)),
          pl.BlockSpec((NBLK, f), lambda i: (i, 0)),
      ],
      out_specs=[
          pl.BlockSpec((2 * heads, NBLK), lambda i: (0, i)),
          pl.BlockSpec((2 * heads, 128), lambda i: (0, 0)),
      ],
      out_shape=[
          jax.ShapeDtypeStruct((2 * heads, n_pad), jnp.float32),
          jax.ShapeDtypeStruct((2 * heads, 128), jnp.float32),
      ],
  )(w_eff, xp)

  mh = mout[:, 0]
  mb = jnp.broadcast_to(
      (mh[:heads] + mh[heads:])[:, None], (heads, LANES)).reshape(-1)

  edge_flat = edge.reshape(-1)
  eexp, part = _make_k1(num_workers, epw, n_pad, heads)(
      edge_flat, alpha.reshape(-1), mb)

  den = pl.pallas_call(
      _k2_body,
      grid=(grid,),
      in_specs=[pl.BlockSpec((num_workers, heads, NBLK), lambda i: (0, 0, i))],
      out_specs=pl.BlockSpec((heads, NBLK), lambda i: (0, i)),
      out_shape=jax.ShapeDtypeStruct((heads, n_pad), jnp.float32),
  )(part.reshape(num_workers, heads, n_pad))

  att = _make_k3(num_workers, epw, n_pad, heads)(
      edge_flat, eexp, den.reshape(-1))
  return att.reshape(e_total, heads)
